# baseline (speedup 1.0000x reference)
import functools

import jax
import jax.numpy as jnp
from jax import lax
from jax.experimental import pallas as pl
from jax.experimental.pallas import tpu as pltpu
from jax.experimental.pallas import tpu_sc as plsc

N = 10000
B = 64
NC = 2
NS = 16
RB = 400
NRB = N // RB
RPS = 624


def _rows_split(s, copy):
    @pl.when(s < NS - 1)
    def _():
        copy(pl.multiple_of(s * RPS, 8), RPS)

    @pl.when(s == NS - 1)
    def _():
        copy((NS - 1) * RPS, N - (NS - 1) * RPS)


def _l1_body(x_ref, w_ref, deg_ref, y_ref):
    dis = lax.rsqrt(deg_ref[:, 0:1] + 1.0)
    h = jnp.dot(x_ref[...], w_ref[...], preferred_element_type=jnp.float32)
    y_ref[...] = dis * h


def _mid_body(z_ref, y_ref, deg_ref, b_ref, w_ref, o_ref):
    dis = lax.rsqrt(deg_ref[:, 0:1] + 1.0)
    z = z_ref[0] + z_ref[1] + y_ref[...]
    h = jnp.maximum(dis * z + b_ref[...], 0.0)
    o_ref[...] = dis * jnp.dot(h, w_ref[...], preferred_element_type=jnp.float32)


def _l3_body(z_ref, y_ref, deg_ref, b_ref, w_ref, oa_ref, ob_ref):
    dis = lax.rsqrt(deg_ref[:, 0:1] + 1.0)
    z = z_ref[0] + z_ref[1] + y_ref[...]
    h = jnp.maximum(dis * z + b_ref[...], 0.0)
    y3 = dis * jnp.dot(h, w_ref[...], preferred_element_type=jnp.float32)
    oa_ref[...] = y3[:, :128]
    ob_ref[...] = y3[:, 128:]


def _pool_body(z_ref, ya_ref, yb_ref, deg_ref, b_ref, bat_ref, out_ref,
               acc_ref, cnt_ref):
    i = pl.program_id(0)

    @pl.when(i == 0)
    def _():
        acc_ref[...] = jnp.zeros_like(acc_ref)
        cnt_ref[...] = jnp.zeros_like(cnt_ref)

    dis = lax.rsqrt(deg_ref[:, 0:1] + 1.0)
    ha = jnp.maximum(dis * (z_ref[0] + ya_ref[...]) + b_ref[:, :128], 0.0)
    hb = jnp.maximum(dis * (z_ref[1] + yb_ref[...]) + b_ref[:, 128:], 0.0)
    oh = (bat_ref[...] == lax.broadcasted_iota(jnp.int32, (RB, B), 1))
    oh = oh.astype(jnp.float32)
    dn = (((0,), (0,)), ((), ()))
    acc_ref[:, :128] += lax.dot_general(oh, ha, dn,
                                        preferred_element_type=jnp.float32)
    acc_ref[:, 128:] += lax.dot_general(oh, hb, dn,
                                        preferred_element_type=jnp.float32)
    cnt_ref[...] += lax.dot_general(oh, jnp.ones((RB, 128), jnp.float32), dn,
                                    preferred_element_type=jnp.float32)

    @pl.when(i == NRB - 1)
    def _():
        m = jnp.maximum(cnt_ref[...], 1.0)
        out_ref[:, :128] = acc_ref[:, :128] / m
        out_ref[:, 128:] = acc_ref[:, 128:] / m


def _mlp_body(pp_ref, pl_ref, w1_ref, b1_ref, w2_ref, b2_ref, wo_ref, bo_ref,
              out_ref):
    x = jnp.concatenate([pp_ref[...], pl_ref[...]], axis=1)
    a = jnp.maximum(jnp.dot(x, w1_ref[...],
                            preferred_element_type=jnp.float32) + b1_ref[...],
                    0.0)
    a = jnp.maximum(jnp.dot(a, w2_ref[...],
                            preferred_element_type=jnp.float32) + b2_ref[...],
                    0.0)
    out_ref[...] = jnp.dot(a, wo_ref[...],
                           preferred_element_type=jnp.float32) + bo_ref[...]


def _row_spec(f):
    return pl.BlockSpec((RB, f), lambda i: (i, 0))


def _full_spec(shape):
    return pl.BlockSpec(shape, lambda i: tuple(0 for _ in shape))


def _tc_l1(x, w, deg):
    fo = w.shape[1]
    return pl.pallas_call(
        _l1_body,
        grid=(NRB,),
        in_specs=[_row_spec(x.shape[1]), _full_spec(w.shape), _row_spec(8)],
        out_specs=_row_spec(fo),
        out_shape=jax.ShapeDtypeStruct((N, fo), jnp.float32),
    )(x, w, deg)


def _tc_mid(z, y, deg, b, w):
    f = y.shape[1]
    fo = w.shape[1]
    return pl.pallas_call(
        _mid_body,
        grid=(NRB,),
        in_specs=[pl.BlockSpec((2, RB, f), lambda i: (0, i, 0)),
                  _row_spec(f), _row_spec(8), _full_spec(b.shape),
                  _full_spec(w.shape)],
        out_specs=_row_spec(fo),
        out_shape=jax.ShapeDtypeStruct((N, fo), jnp.float32),
    )(z, y, deg, b, w)


def _tc_l3(z, y, deg, b, w):
    return pl.pallas_call(
        _l3_body,
        grid=(NRB,),
        in_specs=[pl.BlockSpec((2, RB, 128), lambda i: (0, i, 0)),
                  _row_spec(128), _row_spec(8), _full_spec(b.shape),
                  _full_spec(w.shape)],
        out_specs=[_row_spec(128), _row_spec(128)],
        out_shape=[jax.ShapeDtypeStruct((N, 128), jnp.float32),
                   jax.ShapeDtypeStruct((N, 128), jnp.float32)],
    )(z, y, deg, b, w)


def _tc_pool(z, ya, yb, deg, b, batch2d):
    return pl.pallas_call(
        _pool_body,
        grid=(NRB,),
        in_specs=[pl.BlockSpec((2, RB, 128), lambda i: (0, i, 0)),
                  _row_spec(128), _row_spec(128), _row_spec(8),
                  _full_spec(b.shape), _row_spec(1)],
        out_specs=_full_spec((B, 256)),
        out_shape=jax.ShapeDtypeStruct((B, 256), jnp.float32),
        scratch_shapes=[pltpu.VMEM((B, 256), jnp.float32),
                        pltpu.VMEM((B, 128), jnp.float32)],
    )(z, ya, yb, deg, b, batch2d)


def _tc_mlp(pp, pq, w1, b1, w2, b2, wo, bo):
    return pl.pallas_call(
        _mlp_body,
        out_shape=jax.ShapeDtypeStruct((B, 1), jnp.float32),
    )(pp, pq, w1, b1, w2, b2, wo, bo)


_MESH = plsc.VectorSubcoreMesh(core_axis_name="c", subcore_axis_name="s",
                               num_cores=NC, num_subcores=NS)

NA = N + 8
K = 80


def _start_gather(table, idxrow, buf, sem):
    return pltpu.async_copy(table.at[idxrow], buf, sem)


def _wait_gather(table, idxrow, buf, sem):
    pltpu.make_async_copy(table.at[idxrow], buf, sem).wait()


SB = 64


def _phased_scatter(y_hbm, src_hbm, widx, dst_hbm, acc,
                    srcv, dstv, rows0, rows1, sem0, sem1, ns):
    def phase(p, carry):
        off = pl.multiple_of(p * SB, 8)
        pltpu.sync_copy(src_hbm.at[widx, pl.ds(off, SB)], srcv)
        pltpu.sync_copy(dst_hbm.at[widx, pl.ds(off, SB)], dstv)
        def body(j, c2):
            _start_gather(y_hbm, srcv.at[j], rows0, sem0).wait()
            pltpu.sync_copy(rows0, acc.at[dstv.at[j]], add=True)
            return c2

        lax.fori_loop(0, SB, body, 0)
        return carry

    lax.fori_loop(0, ns // SB, phase, 0)


def _sc_degrees(dstp3d, dstl3d, zeros8, ones8):
    _, ns_p, kp = dstp3d.shape
    _, ns_l, kl = dstl3d.shape

    @functools.partial(
        pl.kernel,
        out_type=(jax.ShapeDtypeStruct((N, 8), jnp.float32),
                  jax.ShapeDtypeStruct((N, 8), jnp.float32)),
        mesh=_MESH,
        compiler_params=pltpu.CompilerParams(use_tc_tiling_on_sc=False),
        scratch_types=[pltpu.VMEM_SHARED((NA, 8), jnp.float32),
                       pltpu.VMEM((ns_p, kp), jnp.int32),
                       pltpu.VMEM((kp, 8), jnp.float32)],
    )
    def k(dstp_hbm, dstl_hbm, z_hbm, ones_hbm, degp_hbm, degl_hbm,
          acc, idx_v, ones_v):
        c = lax.axis_index("c")
        s = lax.axis_index("s")
        _rows_split(s, lambda o, m: pltpu.sync_copy(
            z_hbm.at[pl.ds(o, m)], acc.at[pl.ds(o, m)]))
        pltpu.sync_copy(ones_hbm, ones_v)
        plsc.subcore_barrier()

        @pl.when(c == 0)
        def _():
            pltpu.sync_copy(dstp_hbm.at[s], idx_v)

            def body(j, carry):
                pltpu.sync_copy(ones_v, acc.at[idx_v.at[j]], add=True)
                return carry
            lax.fori_loop(0, ns_p, body, 0)

        @pl.when(c == 1)
        def _():
            pltpu.sync_copy(dstl_hbm.at[s], idx_v.at[pl.ds(0, ns_l)])

            def body(j, carry):
                pltpu.sync_copy(ones_v, acc.at[idx_v.at[j]], add=True)
                return carry
            lax.fori_loop(0, ns_l, body, 0)

        plsc.subcore_barrier()

        @pl.when(c == 0)
        def _():
            _rows_split(s, lambda o, m: pltpu.sync_copy(
                acc.at[pl.ds(o, m)], degp_hbm.at[pl.ds(o, m)]))

        @pl.when(c == 1)
        def _():
            _rows_split(s, lambda o, m: pltpu.sync_copy(
                acc.at[pl.ds(o, m)], degl_hbm.at[pl.ds(o, m)]))

    return k(dstp3d, dstl3d, zeros8, ones8)


def _sc_prop_edge_split(y, src3d, dst3d, zeros):
    f = y.shape[1]
    _, ns, k = src3d.shape

    @functools.partial(
        pl.kernel,
        out_type=jax.ShapeDtypeStruct((NC, N, f), jnp.float32),
        mesh=_MESH,
        compiler_params=pltpu.CompilerParams(use_tc_tiling_on_sc=False),
        scratch_types=[pltpu.VMEM_SHARED((NA, f), jnp.float32),
                       pltpu.VMEM((SB, k), jnp.int32),
                       pltpu.VMEM((SB, k), jnp.int32),
                       pltpu.VMEM((k, f), jnp.float32),
                       pltpu.VMEM((k, f), jnp.float32),
                       pltpu.SemaphoreType.DMA,
                       pltpu.SemaphoreType.DMA],
    )
    def kern(y_hbm, src_hbm, dst_hbm, z_hbm, out_hbm,
             acc, srcv, dstv, rows0, rows1, sem0, sem1):
        c = lax.axis_index("c")
        s = lax.axis_index("s")
        _rows_split(s, lambda o, m: pltpu.sync_copy(
            z_hbm.at[pl.ds(o, m)], acc.at[pl.ds(o, m)]))
        wid = c * NS + s
        plsc.subcore_barrier()
        _phased_scatter(y_hbm, src_hbm, wid, dst_hbm, acc,
                        srcv, dstv, rows0, rows1, sem0, sem1, ns)
        plsc.subcore_barrier()
        _rows_split(s, lambda o, m: pltpu.sync_copy(
            acc.at[pl.ds(o, m)], out_hbm.at[c, pl.ds(o, m)]))

    return kern(y, src3d, dst3d, zeros)


def _sc_prop_feat_split(ya, yb, src3d, dst3d, zeros):
    _, ns, k = src3d.shape

    @functools.partial(
        pl.kernel,
        out_type=jax.ShapeDtypeStruct((NC, N, 128), jnp.float32),
        mesh=_MESH,
        compiler_params=pltpu.CompilerParams(use_tc_tiling_on_sc=False),
        scratch_types=[pltpu.VMEM_SHARED((NA, 128), jnp.float32),
                       pltpu.VMEM((SB, k), jnp.int32),
                       pltpu.VMEM((SB, k), jnp.int32),
                       pltpu.VMEM((k, 128), jnp.float32),
                       pltpu.VMEM((k, 128), jnp.float32),
                       pltpu.SemaphoreType.DMA,
                       pltpu.SemaphoreType.DMA],
    )
    def kern(ya_hbm, yb_hbm, src_hbm, dst_hbm, z_hbm, out_hbm,
             acc, srcv, dstv, rows0, rows1, sem0, sem1):
        c = lax.axis_index("c")
        s = lax.axis_index("s")
        _rows_split(s, lambda o, m: pltpu.sync_copy(
            z_hbm.at[pl.ds(o, m)], acc.at[pl.ds(o, m)]))
        plsc.subcore_barrier()

        @pl.when(c == 0)
        def _():
            _phased_scatter(ya_hbm, src_hbm, s, dst_hbm, acc,
                            srcv, dstv, rows0, rows1, sem0, sem1, ns)

        @pl.when(c == 1)
        def _():
            _phased_scatter(yb_hbm, src_hbm, s, dst_hbm, acc,
                            srcv, dstv, rows0, rows1, sem0, sem1, ns)

        plsc.subcore_barrier()
        _rows_split(s, lambda o, m: pltpu.sync_copy(
            acc.at[pl.ds(o, m)], out_hbm.at[c, pl.ds(o, m)]))

    return kern(ya, yb, src3d, dst3d, zeros)


def _pad_edges(ei, e_pad):
    e = ei.shape[1]
    src = jnp.concatenate([ei[0], jnp.zeros((e_pad - e,), ei.dtype)])
    dst = jnp.concatenate([ei[1], jnp.full((e_pad - e,), N, ei.dtype)])
    return src, dst


def _branch(x, src, dst, deg, W1, b1, W2, b2, W3, b3, zeros64, zeros128):
    s12 = src.reshape(NC * NS, -1, K)
    d12 = dst.reshape(NC * NS, -1, K)
    s80 = src.reshape(NS, -1, K)
    d80 = dst.reshape(NS, -1, K)

    y1 = _tc_l1(x, W1, deg)
    z1 = _sc_prop_edge_split(y1, s12, d12, zeros64)
    y2 = _tc_mid(z1, y1, deg, b1, W2)
    z2 = _sc_prop_edge_split(y2, s12, d12, zeros128)
    y3a, y3b = _tc_l3(z2, y2, deg, b2, W3)
    z3 = _sc_prop_feat_split(y3a, y3b, s80, d80, zeros128)
    return z3, y3a, y3b


def kernel(protein_x, ligand_x, Wp1, bp1, Wp2, bp2, Wp3, bp3, Wl1, bl1, Wl2,
           bl2, Wl3, bl3, Wf1, bf1, Wf2, bf2, Wo, bo, protein_edge_index,
           protein_x_batch, ligand_edge_index, ligand_x_batch):
    zeros8 = jnp.zeros((N, 8), jnp.float32)
    ones8 = jnp.ones((80, 8), jnp.float32)
    zeros64 = jnp.zeros((N, 64), jnp.float32)
    zeros128 = jnp.zeros((N, 128), jnp.float32)

    srcp, dstp = _pad_edges(protein_edge_index, 327680)
    srcl, dstl = _pad_edges(ligand_edge_index, 163840)

    degp, degl = _sc_degrees(
        dstp.reshape(NS, -1, K),
        dstl.reshape(NS, -1, K),
        zeros8, ones8)

    bp1r = bp1.reshape(1, -1)
    bp2r = bp2.reshape(1, -1)
    bp3r = bp3.reshape(1, -1)
    bl1r = bl1.reshape(1, -1)
    bl2r = bl2.reshape(1, -1)
    bl3r = bl3.reshape(1, -1)

    z3p, y3pa, y3pb = _branch(protein_x, srcp, dstp, degp,
                              Wp1, bp1r, Wp2, bp2r, Wp3, bp3r,
                              zeros64, zeros128)
    z3l, y3la, y3lb = _branch(ligand_x, srcl, dstl, degl,
                              Wl1, bl1r, Wl2, bl2r, Wl3, bl3r,
                              zeros64, zeros128)

    pp = _tc_pool(z3p, y3pa, y3pb, degp, bp3r,
                  protein_x_batch.reshape(-1, 1))
    pq = _tc_pool(z3l, y3la, y3lb, degl, bl3r,
                  ligand_x_batch.reshape(-1, 1))

    return _tc_mlp(pp, pq, Wf1, bf1.reshape(1, -1), Wf2, bf2.reshape(1, -1),
                   Wo, bo.reshape(1, -1))


# speedup vs baseline: 3.0516x; 3.0516x over previous
import functools

import jax
import jax.numpy as jnp
from jax import lax
from jax.experimental import pallas as pl
from jax.experimental.pallas import tpu as pltpu
from jax.experimental.pallas import tpu_sc as plsc

N = 10000
B = 64
NC = 2
NS = 16
RB = 400
NRB = N // RB
RPS = 624


def _rows_split(s, copy):
    @pl.when(s < NS - 1)
    def _():
        copy(pl.multiple_of(s * RPS, 8), RPS)

    @pl.when(s == NS - 1)
    def _():
        copy((NS - 1) * RPS, N - (NS - 1) * RPS)


def _l1_body(x_ref, w_ref, deg_ref, y_ref):
    dis = lax.rsqrt(deg_ref[:, 0:1] + 1.0)
    h = jnp.dot(x_ref[...], w_ref[...], preferred_element_type=jnp.float32)
    y_ref[...] = dis * h


def _mid_body(z_ref, y_ref, deg_ref, b_ref, w_ref, o_ref):
    dis = lax.rsqrt(deg_ref[:, 0:1] + 1.0)
    z = z_ref[0] + z_ref[1] + y_ref[...]
    h = jnp.maximum(dis * z + b_ref[...], 0.0)
    o_ref[...] = dis * jnp.dot(h, w_ref[...], preferred_element_type=jnp.float32)


def _l3_body(z_ref, y_ref, deg_ref, b_ref, w_ref, oa_ref, ob_ref):
    dis = lax.rsqrt(deg_ref[:, 0:1] + 1.0)
    z = z_ref[0] + z_ref[1] + y_ref[...]
    h = jnp.maximum(dis * z + b_ref[...], 0.0)
    y3 = dis * jnp.dot(h, w_ref[...], preferred_element_type=jnp.float32)
    oa_ref[...] = y3[:, :128]
    ob_ref[...] = y3[:, 128:]


def _pool_body(z_ref, ya_ref, yb_ref, deg_ref, b_ref, bat_ref, out_ref,
               acc_ref, cnt_ref):
    i = pl.program_id(0)

    @pl.when(i == 0)
    def _():
        acc_ref[...] = jnp.zeros_like(acc_ref)
        cnt_ref[...] = jnp.zeros_like(cnt_ref)

    dis = lax.rsqrt(deg_ref[:, 0:1] + 1.0)
    ha = jnp.maximum(dis * (z_ref[0] + ya_ref[...]) + b_ref[:, :128], 0.0)
    hb = jnp.maximum(dis * (z_ref[1] + yb_ref[...]) + b_ref[:, 128:], 0.0)
    oh = (bat_ref[...] == lax.broadcasted_iota(jnp.int32, (RB, B), 1))
    oh = oh.astype(jnp.float32)
    dn = (((0,), (0,)), ((), ()))
    acc_ref[:, :128] += lax.dot_general(oh, ha, dn,
                                        preferred_element_type=jnp.float32)
    acc_ref[:, 128:] += lax.dot_general(oh, hb, dn,
                                        preferred_element_type=jnp.float32)
    cnt_ref[...] += lax.dot_general(oh, jnp.ones((RB, 128), jnp.float32), dn,
                                    preferred_element_type=jnp.float32)

    @pl.when(i == NRB - 1)
    def _():
        m = jnp.maximum(cnt_ref[...], 1.0)
        out_ref[:, :128] = acc_ref[:, :128] / m
        out_ref[:, 128:] = acc_ref[:, 128:] / m


def _mlp_body(pp_ref, pl_ref, w1_ref, b1_ref, w2_ref, b2_ref, wo_ref, bo_ref,
              out_ref):
    x = jnp.concatenate([pp_ref[...], pl_ref[...]], axis=1)
    a = jnp.maximum(jnp.dot(x, w1_ref[...],
                            preferred_element_type=jnp.float32) + b1_ref[...],
                    0.0)
    a = jnp.maximum(jnp.dot(a, w2_ref[...],
                            preferred_element_type=jnp.float32) + b2_ref[...],
                    0.0)
    out_ref[...] = jnp.dot(a, wo_ref[...],
                           preferred_element_type=jnp.float32) + bo_ref[...]


def _row_spec(f):
    return pl.BlockSpec((RB, f), lambda i: (i, 0))


def _full_spec(shape):
    return pl.BlockSpec(shape, lambda i: tuple(0 for _ in shape))


def _tc_l1(x, w, deg):
    fo = w.shape[1]
    return pl.pallas_call(
        _l1_body,
        grid=(NRB,),
        in_specs=[_row_spec(x.shape[1]), _full_spec(w.shape), _row_spec(8)],
        out_specs=_row_spec(fo),
        out_shape=jax.ShapeDtypeStruct((N, fo), jnp.float32),
    )(x, w, deg)


def _tc_mid(z, y, deg, b, w):
    f = y.shape[1]
    fo = w.shape[1]
    return pl.pallas_call(
        _mid_body,
        grid=(NRB,),
        in_specs=[pl.BlockSpec((2, RB, f), lambda i: (0, i, 0)),
                  _row_spec(f), _row_spec(8), _full_spec(b.shape),
                  _full_spec(w.shape)],
        out_specs=_row_spec(fo),
        out_shape=jax.ShapeDtypeStruct((N, fo), jnp.float32),
    )(z, y, deg, b, w)


def _tc_l3(z, y, deg, b, w):
    return pl.pallas_call(
        _l3_body,
        grid=(NRB,),
        in_specs=[pl.BlockSpec((2, RB, 128), lambda i: (0, i, 0)),
                  _row_spec(128), _row_spec(8), _full_spec(b.shape),
                  _full_spec(w.shape)],
        out_specs=[_row_spec(128), _row_spec(128)],
        out_shape=[jax.ShapeDtypeStruct((N, 128), jnp.float32),
                   jax.ShapeDtypeStruct((N, 128), jnp.float32)],
    )(z, y, deg, b, w)


def _tc_pool(z, ya, yb, deg, b, batch2d):
    return pl.pallas_call(
        _pool_body,
        grid=(NRB,),
        in_specs=[pl.BlockSpec((2, RB, 128), lambda i: (0, i, 0)),
                  _row_spec(128), _row_spec(128), _row_spec(8),
                  _full_spec(b.shape), _row_spec(1)],
        out_specs=_full_spec((B, 256)),
        out_shape=jax.ShapeDtypeStruct((B, 256), jnp.float32),
        scratch_shapes=[pltpu.VMEM((B, 256), jnp.float32),
                        pltpu.VMEM((B, 128), jnp.float32)],
    )(z, ya, yb, deg, b, batch2d)


def _tc_mlp(pp, pq, w1, b1, w2, b2, wo, bo):
    return pl.pallas_call(
        _mlp_body,
        out_shape=jax.ShapeDtypeStruct((B, 1), jnp.float32),
    )(pp, pq, w1, b1, w2, b2, wo, bo)


_MESH = plsc.VectorSubcoreMesh(core_axis_name="c", subcore_axis_name="s",
                               num_cores=NC, num_subcores=NS)

NPAD = 2048
NA = N + NPAD
K = 80


def _start_gather(table, idxrow, buf, sem):
    return pltpu.async_copy(table.at[idxrow], buf, sem)


def _wait_gather(table, idxrow, buf, sem):
    pltpu.make_async_copy(table.at[idxrow], buf, sem).wait()


SB = 64


def _phased_scatter(y_hbm, src_hbm, widx, dst_hbm, acc,
                    srcv, dstv, rows0, rows1, sem0, sem1, ns):
    def phase(p, carry):
        off = pl.multiple_of(p * SB, 8)
        pltpu.sync_copy(src_hbm.at[widx, pl.ds(off, SB)], srcv)
        pltpu.sync_copy(dst_hbm.at[widx, pl.ds(off, SB)], dstv)
        _start_gather(y_hbm, srcv.at[0], rows0, sem0)

        def body(jj, c2):
            j = jj * 2
            _start_gather(y_hbm, srcv.at[j + 1], rows1, sem1)
            _wait_gather(y_hbm, srcv.at[j], rows0, sem0)
            pltpu.sync_copy(rows0, acc.at[dstv.at[j]], add=True)

            @pl.when(j + 2 < SB)
            def _():
                _start_gather(y_hbm, srcv.at[j + 2], rows0, sem0)

            _wait_gather(y_hbm, srcv.at[j + 1], rows1, sem1)
            pltpu.sync_copy(rows1, acc.at[dstv.at[j + 1]], add=True)
            return c2

        lax.fori_loop(0, SB // 2, body, 0)
        return carry

    lax.fori_loop(0, ns // SB, phase, 0)


def _sc_degrees(dstp3d, dstl3d, zeros8, ones8):
    _, ns_p, kp = dstp3d.shape
    _, ns_l, kl = dstl3d.shape

    @functools.partial(
        pl.kernel,
        out_type=(jax.ShapeDtypeStruct((N, 8), jnp.float32),
                  jax.ShapeDtypeStruct((N, 8), jnp.float32)),
        mesh=_MESH,
        compiler_params=pltpu.CompilerParams(use_tc_tiling_on_sc=False),
        scratch_types=[pltpu.VMEM_SHARED((NA, 8), jnp.float32),
                       pltpu.VMEM((ns_p, kp), jnp.int32),
                       pltpu.VMEM((kp, 8), jnp.float32)],
    )
    def k(dstp_hbm, dstl_hbm, z_hbm, ones_hbm, degp_hbm, degl_hbm,
          acc, idx_v, ones_v):
        c = lax.axis_index("c")
        s = lax.axis_index("s")
        _rows_split(s, lambda o, m: pltpu.sync_copy(
            z_hbm.at[pl.ds(o, m)], acc.at[pl.ds(o, m)]))
        pltpu.sync_copy(ones_hbm, ones_v)
        plsc.subcore_barrier()

        @pl.when(c == 0)
        def _():
            pltpu.sync_copy(dstp_hbm.at[s], idx_v)

            def body(j, carry):
                pltpu.sync_copy(ones_v, acc.at[idx_v.at[j]], add=True)
                return carry
            lax.fori_loop(0, ns_p, body, 0)

        @pl.when(c == 1)
        def _():
            pltpu.sync_copy(dstl_hbm.at[s], idx_v.at[pl.ds(0, ns_l)])

            def body(j, carry):
                pltpu.sync_copy(ones_v, acc.at[idx_v.at[j]], add=True)
                return carry
            lax.fori_loop(0, ns_l, body, 0)

        plsc.subcore_barrier()

        @pl.when(c == 0)
        def _():
            _rows_split(s, lambda o, m: pltpu.sync_copy(
                acc.at[pl.ds(o, m)], degp_hbm.at[pl.ds(o, m)]))

        @pl.when(c == 1)
        def _():
            _rows_split(s, lambda o, m: pltpu.sync_copy(
                acc.at[pl.ds(o, m)], degl_hbm.at[pl.ds(o, m)]))

    return k(dstp3d, dstl3d, zeros8, ones8)


def _sc_prop_edge_split(y, src3d, dst3d, zeros):
    f = y.shape[1]
    _, ns, k = src3d.shape

    @functools.partial(
        pl.kernel,
        out_type=jax.ShapeDtypeStruct((NC, N, f), jnp.float32),
        mesh=_MESH,
        compiler_params=pltpu.CompilerParams(use_tc_tiling_on_sc=False),
        scratch_types=[pltpu.VMEM_SHARED((NA, f), jnp.float32),
                       pltpu.VMEM((SB, k), jnp.int32),
                       pltpu.VMEM((SB, k), jnp.int32),
                       pltpu.VMEM((k, f), jnp.float32),
                       pltpu.VMEM((k, f), jnp.float32),
                       pltpu.SemaphoreType.DMA,
                       pltpu.SemaphoreType.DMA],
    )
    def kern(y_hbm, src_hbm, dst_hbm, z_hbm, out_hbm,
             acc, srcv, dstv, rows0, rows1, sem0, sem1):
        c = lax.axis_index("c")
        s = lax.axis_index("s")
        _rows_split(s, lambda o, m: pltpu.sync_copy(
            z_hbm.at[pl.ds(o, m)], acc.at[pl.ds(o, m)]))
        wid = c * NS + s
        plsc.subcore_barrier()
        _phased_scatter(y_hbm, src_hbm, wid, dst_hbm, acc,
                        srcv, dstv, rows0, rows1, sem0, sem1, ns)
        plsc.subcore_barrier()
        _rows_split(s, lambda o, m: pltpu.sync_copy(
            acc.at[pl.ds(o, m)], out_hbm.at[c, pl.ds(o, m)]))

    return kern(y, src3d, dst3d, zeros)


def _sc_prop_feat_split(ya, yb, src3d, dst3d, zeros):
    _, ns, k = src3d.shape

    @functools.partial(
        pl.kernel,
        out_type=jax.ShapeDtypeStruct((NC, N, 128), jnp.float32),
        mesh=_MESH,
        compiler_params=pltpu.CompilerParams(use_tc_tiling_on_sc=False),
        scratch_types=[pltpu.VMEM_SHARED((NA, 128), jnp.float32),
                       pltpu.VMEM((SB, k), jnp.int32),
                       pltpu.VMEM((SB, k), jnp.int32),
                       pltpu.VMEM((k, 128), jnp.float32),
                       pltpu.VMEM((k, 128), jnp.float32),
                       pltpu.SemaphoreType.DMA,
                       pltpu.SemaphoreType.DMA],
    )
    def kern(ya_hbm, yb_hbm, src_hbm, dst_hbm, z_hbm, out_hbm,
             acc, srcv, dstv, rows0, rows1, sem0, sem1):
        c = lax.axis_index("c")
        s = lax.axis_index("s")
        _rows_split(s, lambda o, m: pltpu.sync_copy(
            z_hbm.at[pl.ds(o, m)], acc.at[pl.ds(o, m)]))
        plsc.subcore_barrier()

        @pl.when(c == 0)
        def _():
            _phased_scatter(ya_hbm, src_hbm, s, dst_hbm, acc,
                            srcv, dstv, rows0, rows1, sem0, sem1, ns)

        @pl.when(c == 1)
        def _():
            _phased_scatter(yb_hbm, src_hbm, s, dst_hbm, acc,
                            srcv, dstv, rows0, rows1, sem0, sem1, ns)

        plsc.subcore_barrier()
        _rows_split(s, lambda o, m: pltpu.sync_copy(
            acc.at[pl.ds(o, m)], out_hbm.at[c, pl.ds(o, m)]))

    return kern(ya, yb, src3d, dst3d, zeros)


def _pad_edges(ei, e_pad):
    e = ei.shape[1]
    fill = jnp.arange(e_pad - e, dtype=ei.dtype)
    src = jnp.concatenate([ei[0], fill % N])
    dst = jnp.concatenate([ei[1], N + (fill % NPAD)])
    return src, dst


def _branch(x, src, dst, deg, W1, b1, W2, b2, W3, b3, zeros64, zeros128):
    s12 = src.reshape(NC * NS, -1, K)
    d12 = dst.reshape(NC * NS, -1, K)
    s80 = src.reshape(NS, -1, K)
    d80 = dst.reshape(NS, -1, K)

    y1 = _tc_l1(x, W1, deg)
    z1 = _sc_prop_edge_split(y1, s12, d12, zeros64)
    y2 = _tc_mid(z1, y1, deg, b1, W2)
    z2 = _sc_prop_edge_split(y2, s12, d12, zeros128)
    y3a, y3b = _tc_l3(z2, y2, deg, b2, W3)
    z3 = _sc_prop_feat_split(y3a, y3b, s80, d80, zeros128)
    return z3, y3a, y3b


def kernel(protein_x, ligand_x, Wp1, bp1, Wp2, bp2, Wp3, bp3, Wl1, bl1, Wl2,
           bl2, Wl3, bl3, Wf1, bf1, Wf2, bf2, Wo, bo, protein_edge_index,
           protein_x_batch, ligand_edge_index, ligand_x_batch):
    zeros8 = jnp.zeros((N, 8), jnp.float32)
    ones8 = jnp.ones((80, 8), jnp.float32)
    zeros64 = jnp.zeros((N, 64), jnp.float32)
    zeros128 = jnp.zeros((N, 128), jnp.float32)

    srcp, dstp = _pad_edges(protein_edge_index, 327680)
    srcl, dstl = _pad_edges(ligand_edge_index, 163840)

    degp, degl = _sc_degrees(
        dstp.reshape(NS, -1, K),
        dstl.reshape(NS, -1, K),
        zeros8, ones8)

    bp1r = bp1.reshape(1, -1)
    bp2r = bp2.reshape(1, -1)
    bp3r = bp3.reshape(1, -1)
    bl1r = bl1.reshape(1, -1)
    bl2r = bl2.reshape(1, -1)
    bl3r = bl3.reshape(1, -1)

    z3p, y3pa, y3pb = _branch(protein_x, srcp, dstp, degp,
                              Wp1, bp1r, Wp2, bp2r, Wp3, bp3r,
                              zeros64, zeros128)
    z3l, y3la, y3lb = _branch(ligand_x, srcl, dstl, degl,
                              Wl1, bl1r, Wl2, bl2r, Wl3, bl3r,
                              zeros64, zeros128)

    pp = _tc_pool(z3p, y3pa, y3pb, degp, bp3r,
                  protein_x_batch.reshape(-1, 1))
    pq = _tc_pool(z3l, y3la, y3lb, degl, bl3r,
                  ligand_x_batch.reshape(-1, 1))

    return _tc_mlp(pp, pq, Wf1, bf1.reshape(1, -1), Wf2, bf2.reshape(1, -1),
                   Wo, bo.reshape(1, -1))


# speedup vs baseline: 3.3847x; 1.1092x over previous
import functools

import jax
import jax.numpy as jnp
from jax import lax
from jax.experimental import pallas as pl
from jax.experimental.pallas import tpu as pltpu
from jax.experimental.pallas import tpu_sc as plsc

N = 10000
B = 64
NC = 2
NS = 16
RB = 400
NRB = N // RB
RPS = 624


def _rows_split(s, copy):
    @pl.when(s < NS - 1)
    def _():
        copy(pl.multiple_of(s * RPS, 8), RPS)

    @pl.when(s == NS - 1)
    def _():
        copy((NS - 1) * RPS, N - (NS - 1) * RPS)


def _l1_body(x_ref, w_ref, deg_ref, y_ref):
    dis = lax.rsqrt(deg_ref[:, 0:1] + 1.0)
    h = jnp.dot(x_ref[...], w_ref[...], preferred_element_type=jnp.float32)
    y_ref[...] = dis * h


def _mid_body(z_ref, y_ref, deg_ref, b_ref, w_ref, o_ref):
    dis = lax.rsqrt(deg_ref[:, 0:1] + 1.0)
    z = z_ref[0] + z_ref[1] + y_ref[...]
    h = jnp.maximum(dis * z + b_ref[...], 0.0)
    o_ref[...] = dis * jnp.dot(h, w_ref[...], preferred_element_type=jnp.float32)


def _l3_body(z_ref, y_ref, deg_ref, b_ref, w_ref, oa_ref, ob_ref):
    dis = lax.rsqrt(deg_ref[:, 0:1] + 1.0)
    z = z_ref[0] + z_ref[1] + y_ref[...]
    h = jnp.maximum(dis * z + b_ref[...], 0.0)
    y3 = dis * jnp.dot(h, w_ref[...], preferred_element_type=jnp.float32)
    oa_ref[...] = y3[:, :128]
    ob_ref[...] = y3[:, 128:]


def _pool_body(z_ref, ya_ref, yb_ref, deg_ref, b_ref, bat_ref, out_ref,
               acc_ref, cnt_ref):
    i = pl.program_id(0)

    @pl.when(i == 0)
    def _():
        acc_ref[...] = jnp.zeros_like(acc_ref)
        cnt_ref[...] = jnp.zeros_like(cnt_ref)

    dis = lax.rsqrt(deg_ref[:, 0:1] + 1.0)
    ha = jnp.maximum(dis * (z_ref[0] + ya_ref[...]) + b_ref[:, :128], 0.0)
    hb = jnp.maximum(dis * (z_ref[1] + yb_ref[...]) + b_ref[:, 128:], 0.0)
    oh = (bat_ref[...] == lax.broadcasted_iota(jnp.int32, (RB, B), 1))
    oh = oh.astype(jnp.float32)
    dn = (((0,), (0,)), ((), ()))
    acc_ref[:, :128] += lax.dot_general(oh, ha, dn,
                                        preferred_element_type=jnp.float32)
    acc_ref[:, 128:] += lax.dot_general(oh, hb, dn,
                                        preferred_element_type=jnp.float32)
    cnt_ref[...] += lax.dot_general(oh, jnp.ones((RB, 128), jnp.float32), dn,
                                    preferred_element_type=jnp.float32)

    @pl.when(i == NRB - 1)
    def _():
        m = jnp.maximum(cnt_ref[...], 1.0)
        out_ref[:, :128] = acc_ref[:, :128] / m
        out_ref[:, 128:] = acc_ref[:, 128:] / m


def _mlp_body(pp_ref, pl_ref, w1_ref, b1_ref, w2_ref, b2_ref, wo_ref, bo_ref,
              out_ref):
    x = jnp.concatenate([pp_ref[...], pl_ref[...]], axis=1)
    a = jnp.maximum(jnp.dot(x, w1_ref[...],
                            preferred_element_type=jnp.float32) + b1_ref[...],
                    0.0)
    a = jnp.maximum(jnp.dot(a, w2_ref[...],
                            preferred_element_type=jnp.float32) + b2_ref[...],
                    0.0)
    out_ref[...] = jnp.dot(a, wo_ref[...],
                           preferred_element_type=jnp.float32) + bo_ref[...]


def _row_spec(f):
    return pl.BlockSpec((RB, f), lambda i: (i, 0))


def _full_spec(shape):
    return pl.BlockSpec(shape, lambda i: tuple(0 for _ in shape))


def _tc_l1(x, w, deg):
    fo = w.shape[1]
    return pl.pallas_call(
        _l1_body,
        grid=(NRB,),
        in_specs=[_row_spec(x.shape[1]), _full_spec(w.shape), _row_spec(8)],
        out_specs=_row_spec(fo),
        out_shape=jax.ShapeDtypeStruct((N, fo), jnp.float32),
    )(x, w, deg)


def _tc_mid(z, y, deg, b, w):
    f = y.shape[1]
    fo = w.shape[1]
    return pl.pallas_call(
        _mid_body,
        grid=(NRB,),
        in_specs=[pl.BlockSpec((2, RB, f), lambda i: (0, i, 0)),
                  _row_spec(f), _row_spec(8), _full_spec(b.shape),
                  _full_spec(w.shape)],
        out_specs=_row_spec(fo),
        out_shape=jax.ShapeDtypeStruct((N, fo), jnp.float32),
    )(z, y, deg, b, w)


def _tc_l3(z, y, deg, b, w):
    return pl.pallas_call(
        _l3_body,
        grid=(NRB,),
        in_specs=[pl.BlockSpec((2, RB, 128), lambda i: (0, i, 0)),
                  _row_spec(128), _row_spec(8), _full_spec(b.shape),
                  _full_spec(w.shape)],
        out_specs=[_row_spec(128), _row_spec(128)],
        out_shape=[jax.ShapeDtypeStruct((N, 128), jnp.float32),
                   jax.ShapeDtypeStruct((N, 128), jnp.float32)],
    )(z, y, deg, b, w)


def _tc_pool(z, ya, yb, deg, b, batch2d):
    return pl.pallas_call(
        _pool_body,
        grid=(NRB,),
        in_specs=[pl.BlockSpec((2, RB, 128), lambda i: (0, i, 0)),
                  _row_spec(128), _row_spec(128), _row_spec(8),
                  _full_spec(b.shape), _row_spec(1)],
        out_specs=_full_spec((B, 256)),
        out_shape=jax.ShapeDtypeStruct((B, 256), jnp.float32),
        scratch_shapes=[pltpu.VMEM((B, 256), jnp.float32),
                        pltpu.VMEM((B, 128), jnp.float32)],
    )(z, ya, yb, deg, b, batch2d)


def _tc_mlp(pp, pq, w1, b1, w2, b2, wo, bo):
    return pl.pallas_call(
        _mlp_body,
        out_shape=jax.ShapeDtypeStruct((B, 1), jnp.float32),
    )(pp, pq, w1, b1, w2, b2, wo, bo)


_MESH = plsc.VectorSubcoreMesh(core_axis_name="c", subcore_axis_name="s",
                               num_cores=NC, num_subcores=NS)

NPAD = 512
NA = N + NPAD
K = 128


def _start_gather(table, idxrow, buf, sem):
    return pltpu.async_copy(table.at[idxrow], buf, sem)


def _wait_gather(table, idxrow, buf, sem):
    pltpu.make_async_copy(table.at[idxrow], buf, sem).wait()


SB = 40


def _phased_scatter(y_hbm, src_hbm, widx, dst_hbm, acc,
                    srcv, dstv, rows0, rows1, sem0, sem1, ns):
    def phase(p, carry):
        off = pl.multiple_of(p * SB, 8)
        pltpu.sync_copy(src_hbm.at[widx, pl.ds(off, SB)], srcv)
        pltpu.sync_copy(dst_hbm.at[widx, pl.ds(off, SB)], dstv)
        _start_gather(y_hbm, srcv.at[0], rows0, sem0)

        def body(jj, c2):
            j = jj * 2
            _start_gather(y_hbm, srcv.at[j + 1], rows1, sem1)
            _wait_gather(y_hbm, srcv.at[j], rows0, sem0)
            pltpu.sync_copy(rows0, acc.at[dstv.at[j]], add=True)

            @pl.when(j + 2 < SB)
            def _():
                _start_gather(y_hbm, srcv.at[j + 2], rows0, sem0)

            _wait_gather(y_hbm, srcv.at[j + 1], rows1, sem1)
            pltpu.sync_copy(rows1, acc.at[dstv.at[j + 1]], add=True)
            return c2

        lax.fori_loop(0, SB // 2, body, 0)
        return carry

    lax.fori_loop(0, ns // SB, phase, 0)


def _sc_degrees(dstp3d, dstl3d, zeros8, ones8):
    _, ns_p, kp = dstp3d.shape
    _, ns_l, kl = dstl3d.shape

    @functools.partial(
        pl.kernel,
        out_type=(jax.ShapeDtypeStruct((N, 8), jnp.float32),
                  jax.ShapeDtypeStruct((N, 8), jnp.float32)),
        mesh=_MESH,
        compiler_params=pltpu.CompilerParams(use_tc_tiling_on_sc=False),
        scratch_types=[pltpu.VMEM_SHARED((NA, 8), jnp.float32),
                       pltpu.VMEM((ns_p, kp), jnp.int32),
                       pltpu.VMEM((kp, 8), jnp.float32)],
    )
    def k(dstp_hbm, dstl_hbm, z_hbm, ones_hbm, degp_hbm, degl_hbm,
          acc, idx_v, ones_v):
        c = lax.axis_index("c")
        s = lax.axis_index("s")
        _rows_split(s, lambda o, m: pltpu.sync_copy(
            z_hbm.at[pl.ds(o, m)], acc.at[pl.ds(o, m)]))
        pltpu.sync_copy(ones_hbm, ones_v)
        plsc.subcore_barrier()

        @pl.when(c == 0)
        def _():
            pltpu.sync_copy(dstp_hbm.at[s], idx_v)

            def body(j, carry):
                pltpu.sync_copy(ones_v, acc.at[idx_v.at[j]], add=True)
                return carry
            lax.fori_loop(0, ns_p, body, 0)

        @pl.when(c == 1)
        def _():
            pltpu.sync_copy(dstl_hbm.at[s], idx_v.at[pl.ds(0, ns_l)])

            def body(j, carry):
                pltpu.sync_copy(ones_v, acc.at[idx_v.at[j]], add=True)
                return carry
            lax.fori_loop(0, ns_l, body, 0)

        plsc.subcore_barrier()

        @pl.when(c == 0)
        def _():
            _rows_split(s, lambda o, m: pltpu.sync_copy(
                acc.at[pl.ds(o, m)], degp_hbm.at[pl.ds(o, m)]))

        @pl.when(c == 1)
        def _():
            _rows_split(s, lambda o, m: pltpu.sync_copy(
                acc.at[pl.ds(o, m)], degl_hbm.at[pl.ds(o, m)]))

    return k(dstp3d, dstl3d, zeros8, ones8)


def _sc_prop_edge_split(y, src3d, dst3d, zeros):
    f = y.shape[1]
    _, ns, k = src3d.shape

    @functools.partial(
        pl.kernel,
        out_type=jax.ShapeDtypeStruct((NC, N, f), jnp.float32),
        mesh=_MESH,
        compiler_params=pltpu.CompilerParams(use_tc_tiling_on_sc=False),
        scratch_types=[pltpu.VMEM_SHARED((NA, f), jnp.float32),
                       pltpu.VMEM((SB, k), jnp.int32),
                       pltpu.VMEM((SB, k), jnp.int32),
                       pltpu.VMEM((k, f), jnp.float32),
                       pltpu.VMEM((k, f), jnp.float32),
                       pltpu.SemaphoreType.DMA,
                       pltpu.SemaphoreType.DMA],
    )
    def kern(y_hbm, src_hbm, dst_hbm, z_hbm, out_hbm,
             acc, srcv, dstv, rows0, rows1, sem0, sem1):
        c = lax.axis_index("c")
        s = lax.axis_index("s")
        _rows_split(s, lambda o, m: pltpu.sync_copy(
            z_hbm.at[pl.ds(o, m)], acc.at[pl.ds(o, m)]))
        wid = c * NS + s
        plsc.subcore_barrier()
        _phased_scatter(y_hbm, src_hbm, wid, dst_hbm, acc,
                        srcv, dstv, rows0, rows1, sem0, sem1, ns)
        plsc.subcore_barrier()
        _rows_split(s, lambda o, m: pltpu.sync_copy(
            acc.at[pl.ds(o, m)], out_hbm.at[c, pl.ds(o, m)]))

    return kern(y, src3d, dst3d, zeros)


def _sc_prop_feat_split(ya, yb, src3d, dst3d, zeros):
    _, ns, k = src3d.shape

    @functools.partial(
        pl.kernel,
        out_type=jax.ShapeDtypeStruct((NC, N, 128), jnp.float32),
        mesh=_MESH,
        compiler_params=pltpu.CompilerParams(use_tc_tiling_on_sc=False),
        scratch_types=[pltpu.VMEM_SHARED((NA, 128), jnp.float32),
                       pltpu.VMEM((SB, k), jnp.int32),
                       pltpu.VMEM((SB, k), jnp.int32),
                       pltpu.VMEM((k, 128), jnp.float32),
                       pltpu.VMEM((k, 128), jnp.float32),
                       pltpu.SemaphoreType.DMA,
                       pltpu.SemaphoreType.DMA],
    )
    def kern(ya_hbm, yb_hbm, src_hbm, dst_hbm, z_hbm, out_hbm,
             acc, srcv, dstv, rows0, rows1, sem0, sem1):
        c = lax.axis_index("c")
        s = lax.axis_index("s")
        _rows_split(s, lambda o, m: pltpu.sync_copy(
            z_hbm.at[pl.ds(o, m)], acc.at[pl.ds(o, m)]))
        plsc.subcore_barrier()

        @pl.when(c == 0)
        def _():
            _phased_scatter(ya_hbm, src_hbm, s, dst_hbm, acc,
                            srcv, dstv, rows0, rows1, sem0, sem1, ns)

        @pl.when(c == 1)
        def _():
            _phased_scatter(yb_hbm, src_hbm, s, dst_hbm, acc,
                            srcv, dstv, rows0, rows1, sem0, sem1, ns)

        plsc.subcore_barrier()
        _rows_split(s, lambda o, m: pltpu.sync_copy(
            acc.at[pl.ds(o, m)], out_hbm.at[c, pl.ds(o, m)]))

    return kern(ya, yb, src3d, dst3d, zeros)


def _pad_edges(ei, e_pad):
    e = ei.shape[1]
    fill = jnp.arange(e_pad - e, dtype=ei.dtype)
    src = jnp.concatenate([ei[0], fill % N])
    dst = jnp.concatenate([ei[1], N + (fill % NPAD)])
    return src, dst


def _branch(x, src, dst, deg, W1, b1, W2, b2, W3, b3, zeros64, zeros128):
    s12 = src.reshape(NC * NS, -1, K)
    d12 = dst.reshape(NC * NS, -1, K)
    s80 = src.reshape(NS, -1, K)
    d80 = dst.reshape(NS, -1, K)

    y1 = _tc_l1(x, W1, deg)
    z1 = _sc_prop_edge_split(y1, s12, d12, zeros64)
    y2 = _tc_mid(z1, y1, deg, b1, W2)
    z2 = _sc_prop_edge_split(y2, s12, d12, zeros128)
    y3a, y3b = _tc_l3(z2, y2, deg, b2, W3)
    z3 = _sc_prop_feat_split(y3a, y3b, s80, d80, zeros128)
    return z3, y3a, y3b


def kernel(protein_x, ligand_x, Wp1, bp1, Wp2, bp2, Wp3, bp3, Wl1, bl1, Wl2,
           bl2, Wl3, bl3, Wf1, bf1, Wf2, bf2, Wo, bo, protein_edge_index,
           protein_x_batch, ligand_edge_index, ligand_x_batch):
    zeros8 = jnp.zeros((N, 8), jnp.float32)
    ones8 = jnp.ones((K, 8), jnp.float32)
    zeros64 = jnp.zeros((N, 64), jnp.float32)
    zeros128 = jnp.zeros((N, 128), jnp.float32)

    srcp, dstp = _pad_edges(protein_edge_index, 327680)
    srcl, dstl = _pad_edges(ligand_edge_index, 163840)

    degp, degl = _sc_degrees(
        dstp.reshape(NS, -1, K),
        dstl.reshape(NS, -1, K),
        zeros8, ones8)

    bp1r = bp1.reshape(1, -1)
    bp2r = bp2.reshape(1, -1)
    bp3r = bp3.reshape(1, -1)
    bl1r = bl1.reshape(1, -1)
    bl2r = bl2.reshape(1, -1)
    bl3r = bl3.reshape(1, -1)

    z3p, y3pa, y3pb = _branch(protein_x, srcp, dstp, degp,
                              Wp1, bp1r, Wp2, bp2r, Wp3, bp3r,
                              zeros64, zeros128)
    z3l, y3la, y3lb = _branch(ligand_x, srcl, dstl, degl,
                              Wl1, bl1r, Wl2, bl2r, Wl3, bl3r,
                              zeros64, zeros128)

    pp = _tc_pool(z3p, y3pa, y3pb, degp, bp3r,
                  protein_x_batch.reshape(-1, 1))
    pq = _tc_pool(z3l, y3la, y3lb, degl, bl3r,
                  ligand_x_batch.reshape(-1, 1))

    return _tc_mlp(pp, pq, Wf1, bf1.reshape(1, -1), Wf2, bf2.reshape(1, -1),
                   Wo, bo.reshape(1, -1))
